# baseline (speedup 1.0000x reference)
import functools
import jax
import jax.numpy as jnp
from jax.experimental import pallas as pl

SEQ_LEN = 512
D_MODEL = 256
D_FF = 512
N_HEADS = 4
BUCKET = 32
N_HASHES = 4
HID = 256
NLAB = 9
TOK = 77
DH = D_MODEL // N_HEADS
NB = SEQ_LEN // BUCKET
NBK = N_HASHES * NB
S = N_HASHES * SEQ_LEN
NCH = S // BUCKET


def _pos_emb():
    import numpy as np
    pe = np.zeros((SEQ_LEN, D_MODEL), np.float32)
    pos = np.arange(SEQ_LEN, dtype=np.float32)[:, None]
    div = np.exp(np.arange(0, D_MODEL, 2, dtype=np.float32) * (-np.log(10000.0) / D_MODEL))
    pe[:, 0::2] = np.sin(pos * div)
    pe[:, 1::2] = np.cos(pos * div)
    return jnp.asarray(pe)


def _embed_body(x_ref, w_ref, pos_ref, o_ref):
    x = x_ref[0]
    xm1 = jnp.concatenate([x[SEQ_LEN - 1:], x[:SEQ_LEN - 1]], axis=0)
    xp1 = jnp.concatenate([x[1:], x[:1]], axis=0)
    o = (jnp.dot(xm1, w_ref[0], preferred_element_type=jnp.float32)
         + jnp.dot(x, w_ref[1], preferred_element_type=jnp.float32)
         + jnp.dot(xp1, w_ref[2], preferred_element_type=jnp.float32))
    o_ref[0] = o + pos_ref[...]


def _embed(x, W, pos):
    Bb, T, Cin = x.shape
    return pl.pallas_call(
        _embed_body,
        grid=(Bb,),
        in_specs=[pl.BlockSpec((1, T, Cin), lambda b: (b, 0, 0)),
                  pl.BlockSpec((3, Cin, D_MODEL), lambda b: (0, 0, 0)),
                  pl.BlockSpec((T, D_MODEL), lambda b: (0, 0))],
        out_specs=pl.BlockSpec((1, T, D_MODEL), lambda b: (b, 0, 0)),
        out_shape=jax.ShapeDtypeStruct((Bb, T, D_MODEL), jnp.float32),
    )(x, W, pos)


def _qkvb_body(h_ref, wqk_ref, wv_ref, rot_ref, qk_ref, v_ref, bc_ref):
    h = h_ref[0]
    qk = jnp.dot(h, wqk_ref[...], preferred_element_type=jnp.float32)
    v = jnp.dot(h, wv_ref[...], preferred_element_type=jnp.float32)
    iota16 = jax.lax.broadcasted_iota(jnp.int32, (SEQ_LEN, NB), 1).astype(jnp.float32)
    for hh in range(N_HEADS):
        qkh = qk[:, hh * DH:(hh + 1) * DH]
        qk_ref[0, hh] = qkh
        v_ref[0, hh] = v[:, hh * DH:(hh + 1) * DH]
        r = jnp.dot(qkh, rot_ref[...], preferred_element_type=jnp.float32)
        for ha in range(N_HASHES):
            sub = r[:, ha * NB:(ha + 1) * NB]
            m = jnp.max(sub, axis=1, keepdims=True)
            idx = jnp.min(jnp.where(sub >= m, iota16, 1e9), axis=1, keepdims=True)
            bc_ref[0, :, hh * N_HASHES + ha:hh * N_HASHES + ha + 1] = idx + float(ha * NB)


def _qkvb(h, Wqk, Wv, rot2p):
    Bb = h.shape[0]
    return pl.pallas_call(
        _qkvb_body,
        grid=(Bb,),
        in_specs=[pl.BlockSpec((1, SEQ_LEN, D_MODEL), lambda b: (b, 0, 0)),
                  pl.BlockSpec((D_MODEL, D_MODEL), lambda b: (0, 0)),
                  pl.BlockSpec((D_MODEL, D_MODEL), lambda b: (0, 0)),
                  pl.BlockSpec((DH, NBK), lambda b: (0, 0))],
        out_specs=[pl.BlockSpec((1, N_HEADS, SEQ_LEN, DH), lambda b: (b, 0, 0, 0)),
                   pl.BlockSpec((1, N_HEADS, SEQ_LEN, DH), lambda b: (b, 0, 0, 0)),
                   pl.BlockSpec((1, SEQ_LEN, N_HEADS * N_HASHES), lambda b: (b, 0, 0))],
        out_shape=[jax.ShapeDtypeStruct((Bb, N_HEADS, SEQ_LEN, DH), jnp.float32),
                   jax.ShapeDtypeStruct((Bb, N_HEADS, SEQ_LEN, DH), jnp.float32),
                   jax.ShapeDtypeStruct((Bb, SEQ_LEN, N_HEADS * N_HASHES), jnp.float32)],
    )(h, Wqk, Wv, rot2p)


def _rank_body(bc_ref, p_ref):
    lt = (jax.lax.broadcasted_iota(jnp.int32, (NBK, NBK), 0)
          < jax.lax.broadcasted_iota(jnp.int32, (NBK, NBK), 1)).astype(jnp.float32)
    iota_l = jax.lax.broadcasted_iota(jnp.int32, (S, NBK), 1).astype(jnp.float32)
    for hh in range(N_HEADS):
        cols = [bc_ref[0, :, hh * N_HASHES + ha:hh * N_HASHES + ha + 1]
                for ha in range(N_HASHES)]
        bcol = jnp.concatenate(cols, axis=0)
        O = (jnp.broadcast_to(bcol, (S, NBK)) == iota_l).astype(jnp.float32)
        csum = O
        sh = 1
        while sh < S:
            shifted = jnp.concatenate(
                [jnp.zeros((sh, NBK), jnp.float32), csum[:S - sh]], axis=0)
            csum = csum + shifted
            sh *= 2
        tot = csum[S - 1:S, :]
        offs = jnp.dot(tot, lt, preferred_element_type=jnp.float32)
        rank = jnp.sum(O * csum, axis=1, keepdims=True) - 1.0
        offj = jnp.sum(O * jnp.broadcast_to(offs, (S, NBK)), axis=1, keepdims=True)
        p_ref[0, hh] = (rank + offj).astype(jnp.int32)


def _rank(bc):
    Bb = bc.shape[0]
    return pl.pallas_call(
        _rank_body,
        grid=(Bb,),
        in_specs=[pl.BlockSpec((1, SEQ_LEN, N_HEADS * N_HASHES), lambda b: (b, 0, 0))],
        out_specs=pl.BlockSpec((1, N_HEADS, S, 1), lambda b: (b, 0, 0, 0)),
        out_shape=jax.ShapeDtypeStruct((Bb, N_HEADS, S, 1), jnp.int32),
    )(bc)


def _attn_body(sqk_ref, sv_ref, tsc_ref, tscr_ref, so_ref, sl_ref):
    def chunk(c, _):
        cur = c * BUCKET
        prv = jnp.where(c == 0, S - BUCKET, cur - BUCKET)
        q = sqk_ref[0, pl.ds(cur, BUCKET), :]
        kp = sqk_ref[0, pl.ds(prv, BUCKET), :]
        k2 = jnp.concatenate([kp, q], axis=0)
        nrm = jnp.sqrt(jnp.sum(k2 * k2, axis=1, keepdims=True))
        k2 = k2 / (nrm + 1e-9)
        v2 = jnp.concatenate([sv_ref[0, pl.ds(prv, BUCKET), :],
                              sv_ref[0, pl.ds(cur, BUCKET), :]], axis=0)
        dots = jax.lax.dot_general(q, k2, (((1,), (1,)), ((), ())),
                                   preferred_element_type=jnp.float32) * (DH ** -0.5)
        tsq = tsc_ref[0, pl.ds(cur, BUCKET), :]
        pc = jnp.where(c == 0, NCH - 1, c - 1)
        tskp = tscr_ref[0, pl.ds(pc, 1), :]
        tskc = tscr_ref[0, pl.ds(c, 1), :]
        tsk = jnp.concatenate([tskp, tskc], axis=1)
        mask = jnp.broadcast_to(tsq, (BUCKET, 2 * BUCKET)) == jnp.broadcast_to(tsk, (BUCKET, 2 * BUCKET))
        dots = jnp.where(mask, -5e4, dots)
        m = jnp.max(dots, axis=1, keepdims=True)
        ex = jnp.exp(dots - m)
        lse = m + jnp.log(jnp.sum(ex, axis=1, keepdims=True))
        pr = jnp.exp(dots - lse)
        o = jnp.dot(pr, v2, preferred_element_type=jnp.float32)
        so_ref[0, pl.ds(cur, BUCKET), :] = o
        sl_ref[0, pl.ds(cur, BUCKET), :] = lse
        return 0

    jax.lax.fori_loop(0, NCH, chunk, 0)


def _attn(sqk, sv, sts):
    BH = sqk.shape[0]
    tsc = sts.reshape(BH, S, 1)
    tscr = sts.reshape(BH, NCH, BUCKET)
    return pl.pallas_call(
        _attn_body,
        grid=(BH,),
        in_specs=[pl.BlockSpec((1, S, DH), lambda b: (b, 0, 0)),
                  pl.BlockSpec((1, S, DH), lambda b: (b, 0, 0)),
                  pl.BlockSpec((1, S, 1), lambda b: (b, 0, 0)),
                  pl.BlockSpec((1, NCH, BUCKET), lambda b: (b, 0, 0))],
        out_specs=[pl.BlockSpec((1, S, DH), lambda b: (b, 0, 0)),
                   pl.BlockSpec((1, S, 1), lambda b: (b, 0, 0))],
        out_shape=[jax.ShapeDtypeStruct((BH, S, DH), jnp.float32),
                   jax.ShapeDtypeStruct((BH, S, 1), jnp.float32)],
    )(sqk, sv, tsc, tscr)


def _ln_in(x, g, b):
    m = jnp.mean(x, axis=1, keepdims=True)
    v = jnp.mean((x - m) ** 2, axis=1, keepdims=True)
    return (x - m) / jnp.sqrt(v + 1e-5) * g + b


def _combine_body(og_ref, lg_ref, h_ref, wo_ref, bo_ref, g1_ref, b1_ref,
                  w1_ref, bf1_ref, w2_ref, bf2_ref, g2_ref, b2_ref, o_ref):
    ao = jnp.zeros((SEQ_LEN, D_MODEL), jnp.float32)
    for hh in range(N_HEADS):
        ls = [lg_ref[0, hh, ha] for ha in range(N_HASHES)]
        m = jnp.maximum(jnp.maximum(ls[0], ls[1]), jnp.maximum(ls[2], ls[3]))
        es = [jnp.exp(l - m) for l in ls]
        se = es[0] + es[1] + es[2] + es[3]
        att = jnp.zeros((SEQ_LEN, DH), jnp.float32)
        for ha in range(N_HASHES):
            att = att + (es[ha] / se) * og_ref[0, hh, ha]
        ao = ao + jnp.dot(att, wo_ref[pl.ds(hh * DH, DH), :],
                          preferred_element_type=jnp.float32)
    h = h_ref[0] + ao + bo_ref[...]
    h = _ln_in(h, g1_ref[...], b1_ref[...])
    y = jax.nn.gelu(jnp.dot(h, w1_ref[...], preferred_element_type=jnp.float32)
                    + bf1_ref[...])
    y = jnp.dot(y, w2_ref[...], preferred_element_type=jnp.float32) + bf2_ref[...]
    o_ref[0] = _ln_in(h + y, g2_ref[...], b2_ref[...])


def _combine(og, lg, h, lp):
    Bb = h.shape[0]
    og5 = og.reshape(Bb, N_HEADS, N_HASHES, SEQ_LEN, DH)
    lg5 = lg.reshape(Bb, N_HEADS, N_HASHES, SEQ_LEN, 1)
    row = lambda n: pl.BlockSpec((1, D_MODEL), lambda b: (0, 0))
    mat = lambda m, n: pl.BlockSpec((m, n), lambda b: (0, 0))
    return pl.pallas_call(
        _combine_body,
        grid=(Bb,),
        in_specs=[pl.BlockSpec((1, N_HEADS, N_HASHES, SEQ_LEN, DH), lambda b: (b, 0, 0, 0, 0)),
                  pl.BlockSpec((1, N_HEADS, N_HASHES, SEQ_LEN, 1), lambda b: (b, 0, 0, 0, 0)),
                  pl.BlockSpec((1, SEQ_LEN, D_MODEL), lambda b: (b, 0, 0)),
                  mat(D_MODEL, D_MODEL), row(D_MODEL), row(D_MODEL), row(D_MODEL),
                  mat(D_MODEL, D_FF), pl.BlockSpec((1, D_FF), lambda b: (0, 0)),
                  mat(D_FF, D_MODEL), row(D_MODEL), row(D_MODEL), row(D_MODEL)],
        out_specs=pl.BlockSpec((1, SEQ_LEN, D_MODEL), lambda b: (b, 0, 0)),
        out_shape=jax.ShapeDtypeStruct((Bb, SEQ_LEN, D_MODEL), jnp.float32),
    )(og5, lg5, h,
      lp['Wo'], lp['bo'].reshape(1, -1), lp['g1'].reshape(1, -1), lp['b1'].reshape(1, -1),
      lp['W1'], lp['bf1'].reshape(1, -1), lp['W2'], lp['bf2'].reshape(1, -1),
      lp['g2'].reshape(1, -1), lp['b2'].reshape(1, -1))


TT = 32


def _proj_body(h_ref, m_ref, gf_ref, bf_ref, wp_ref, bp_ref, o_ref):
    t0 = pl.program_id(0)

    @pl.when(t0 == 0)
    def _init():
        o_ref[...] = jnp.broadcast_to(bp_ref[...], o_ref.shape)

    h = h_ref[...]
    m = jnp.mean(h, axis=2, keepdims=True)
    v = jnp.mean((h - m) ** 2, axis=2, keepdims=True)
    hn = (h - m) / jnp.sqrt(v + 1e-5) * gf_ref[...] + bf_ref[...]
    g = jax.nn.gelu(hn) * m_ref[...]
    acc = jnp.zeros(o_ref.shape, jnp.float32)
    for tt in range(TT):
        acc = acc + jnp.dot(g[:, tt, :], wp_ref[tt],
                            preferred_element_type=jnp.float32)
    o_ref[...] = o_ref[...] + acc


def _proj(h, mask, gf, bf, Wproj, bproj):
    Bb = h.shape[0]
    wp3 = Wproj.reshape(SEQ_LEN, D_MODEL, HID)
    m3 = mask.reshape(Bb, SEQ_LEN, 1)
    nt = SEQ_LEN // TT
    return pl.pallas_call(
        _proj_body,
        grid=(nt,),
        in_specs=[pl.BlockSpec((Bb, TT, D_MODEL), lambda t: (0, t, 0)),
                  pl.BlockSpec((Bb, TT, 1), lambda t: (0, t, 0)),
                  pl.BlockSpec((1, 1, D_MODEL), lambda t: (0, 0, 0)),
                  pl.BlockSpec((1, 1, D_MODEL), lambda t: (0, 0, 0)),
                  pl.BlockSpec((TT, D_MODEL, HID), lambda t: (t, 0, 0)),
                  pl.BlockSpec((1, HID), lambda t: (0, 0))],
        out_specs=pl.BlockSpec((Bb, HID), lambda t: (0, 0)),
        out_shape=jax.ShapeDtypeStruct((Bb, HID), jnp.float32),
    )(h, m3, gf.reshape(1, 1, -1), bf.reshape(1, 1, -1), wp3, bproj.reshape(1, -1))


def _head_body(txt_ref, tm_ref, wt_ref, bt_ref, cv_ref, ca_ref,
               wh1_ref, bh1_ref, wh2_ref, bh2_ref, o_ref):
    Bb = txt_ref.shape[0]
    rows = []
    for b in range(Bb):
        row = tm_ref[pl.ds(b, 1), :]
        cnt = jnp.maximum(jnp.sum(row, axis=1, keepdims=True), 1.0)
        sb = jnp.dot(row, txt_ref[b], preferred_element_type=jnp.float32)
        rows.append(sb / cnt)
    th = jnp.concatenate(rows, axis=0)
    ct = jnp.dot(th, wt_ref[...], preferred_element_type=jnp.float32) + bt_ref[...]
    f = (jnp.dot(cv_ref[...], wh1_ref[pl.ds(0, HID), :], preferred_element_type=jnp.float32)
         + jnp.dot(ca_ref[...], wh1_ref[pl.ds(HID, HID), :], preferred_element_type=jnp.float32)
         + jnp.dot(ct, wh1_ref[pl.ds(2 * HID, HID), :], preferred_element_type=jnp.float32))
    f = jax.nn.relu(f + bh1_ref[...])
    o_ref[...] = jnp.dot(f, wh2_ref[...], preferred_element_type=jnp.float32) + bh2_ref[...]


def _head(txt, tm, Wt, bt, cv, ca, Wh1, bh1, Wh2, bh2):
    Bb, Tk, TX = txt.shape
    return pl.pallas_call(
        _head_body,
        out_shape=jax.ShapeDtypeStruct((Bb, NLAB), jnp.float32),
    )(txt, tm, Wt, bt.reshape(1, -1), cv, ca, Wh1, bh1.reshape(1, -1),
      Wh2, bh2.reshape(1, -1))


def _apply_perm(qk, v, p):
    BH = qk.shape[0]
    bhi = jnp.arange(BH, dtype=jnp.int32)[:, None]
    src_t = jnp.tile(jnp.arange(SEQ_LEN, dtype=jnp.int32), N_HASHES)[None, :]
    sqk = jnp.zeros((BH, S, DH), jnp.float32).at[bhi, p, :].set(
        jnp.take_along_axis(qk, jnp.broadcast_to(src_t[..., None], (BH, S, 1)), axis=1))
    sv = jnp.zeros((BH, S, DH), jnp.float32).at[bhi, p, :].set(
        jnp.take_along_axis(v, jnp.broadcast_to(src_t[..., None], (BH, S, 1)), axis=1))
    sts = jnp.zeros((BH, S), jnp.int32).at[bhi, p].set(
        jnp.broadcast_to(src_t, (BH, S)))
    return sqk, sv, sts


def _unperm(so, sl, p):
    og = jnp.take_along_axis(so, p[..., None], axis=1)
    lg = jnp.take_along_axis(sl[..., 0], p, axis=1)
    return og, lg


def _encoder(x, enc, key, pos):
    Bb = x.shape[0]
    h = _embed(x, enc['Wemb'], pos)
    for i, lp in enumerate(enc['layers']):
        kl = jax.random.fold_in(key, i)
        rot = jax.random.normal(kl, (DH, N_HASHES, NB // 2), dtype=jnp.float32)
        rot2p = jnp.concatenate([rot, -rot], axis=2).reshape(DH, NBK)
        qk, v, bc = _qkvb(h, lp['Wqk'], lp['Wv'], rot2p)
        p = _rank(bc).reshape(Bb * N_HEADS, S)
        qkf = qk.reshape(Bb * N_HEADS, SEQ_LEN, DH)
        vf = v.reshape(Bb * N_HEADS, SEQ_LEN, DH)
        sqk, sv, sts = _apply_perm(qkf, vf, p)
        so, sl = _attn(sqk, sv, sts)
        og, lg = _unperm(so, sl, p)
        h = _combine(og, lg, h, lp)
    return h


def kernel(counselor_video, counselor_audio, counselor_text,
           counselor_word_mask, counselor_tok_mask, params):
    pos = _pos_emb()
    kv = jax.random.key(1234)
    outs = []
    for x, name, ki in ((counselor_video, 'visual', 0),
                        (counselor_audio, 'audio', 1)):
        enc = params[name]
        h = _encoder(x, enc, jax.random.fold_in(kv, ki), pos)
        outs.append(_proj(h, counselor_word_mask, enc['gf'], enc['bf'],
                          enc['Wproj'], enc['bproj']))
    tm = counselor_tok_mask.astype(jnp.float32)
    return _head(counselor_text, tm, params['Wt'], params['bt'],
                 outs[0], outs[1], params['Wh1'], params['bh1'],
                 params['Wh2'], params['bh2'])
